# Initial kernel scaffold; baseline (speedup 1.0000x reference)
import jax
import jax.numpy as jnp
from jax.experimental import pallas as pl


def kernel(aff, sam_masks, topk):
    raise NotImplementedError("write your pallas kernel here")


# speedup vs baseline: 16.6870x; 16.6870x over previous
import functools

import jax
import jax.numpy as jnp
from jax.experimental import pallas as pl
from jax.experimental.pallas import tpu as pltpu

_TEMPERATURE = 100.0


def _block_kernel(topk_ref, aff_ref, sam_ref, iou_ref, best_ref, most_ref,
                  counts_ref):
    x = aff_ref[...]
    rows, n = x.shape
    m = sam_ref.shape[0]

    logits = x / _TEMPERATURE
    mx = jnp.max(logits, axis=-1, keepdims=True)
    e = jnp.exp(logits - mx)
    denom = jnp.sum(e, axis=-1, keepdims=True)
    sm = e / denom

    keys = jax.lax.bitcast_convert_type(sm, jnp.int32)
    k = topk_ref[0] + 1

    def body(i, cur):
        cand = cur + (jnp.int32(1) << (jnp.int32(30) - i))
        ge = (keys >= cand).astype(jnp.int32)
        cnt = jnp.sum(ge, axis=-1, keepdims=True)
        return jnp.where(cnt >= k, cand, cur)

    tkey = jax.lax.fori_loop(0, 31, body,
                             jnp.zeros((rows, 1), jnp.int32))
    thresh = jax.lax.bitcast_convert_type(tkey, jnp.float32)

    sam = sam_ref[...]
    masked = jnp.where(sm > thresh, sm, 0.0)
    inter = jax.lax.dot_general(
        masked, sam, (((1,), (1,)), ((), ())),
        preferred_element_type=jnp.float32)
    sel_sum = jnp.sum(masked, axis=-1, keepdims=True)
    sam_sum = jnp.sum(sam, axis=-1)
    union = sel_sum + sam_sum[None, :] - inter
    iou = inter / (union + 1e-8)
    iou_ref[...] = iou

    iou_mx = jnp.max(iou, axis=-1, keepdims=True)
    lane = jax.lax.broadcasted_iota(jnp.int32, (rows, m), 1)
    best = jnp.min(jnp.where(iou == iou_mx, lane, m), axis=-1)
    best_ref[...] = best[:, None]

    @pl.when(pl.program_id(0) == 0)
    def _init():
        counts_ref[...] = jnp.zeros_like(counts_ref)

    onehot = (best[:, None] == lane).astype(jnp.int32)
    counts_ref[...] += jnp.sum(onehot, axis=0, keepdims=True)

    @pl.when(pl.program_id(0) == pl.num_programs(0) - 1)
    def _finalize():
        c = counts_ref[...]
        c_mx = jnp.max(c, axis=-1, keepdims=True)
        c_lane = jax.lax.broadcasted_iota(jnp.int32, (1, m), 1)
        win = jnp.min(jnp.where(c == c_mx, c_lane, m), axis=-1, keepdims=True)
        most_ref[...] = jnp.broadcast_to(win, (1, m))


def kernel(aff, sam_masks, topk):
    b, n = aff.shape
    m = sam_masks.shape[0]
    block_rows = 256 if b % 256 == 0 else b
    grid = b // block_rows
    topk_arr = jnp.asarray(topk, jnp.int32).reshape(1)

    iou, best2d, most2d = pl.pallas_call(
        _block_kernel,
        grid=(grid,),
        in_specs=[
            pl.BlockSpec(memory_space=pltpu.SMEM),
            pl.BlockSpec((block_rows, n), lambda i: (i, 0)),
            pl.BlockSpec((m, n), lambda i: (0, 0)),
        ],
        out_specs=[
            pl.BlockSpec((block_rows, m), lambda i: (i, 0)),
            pl.BlockSpec((block_rows, 1), lambda i: (i, 0)),
            pl.BlockSpec((1, m), lambda i: (0, 0)),
        ],
        out_shape=[
            jax.ShapeDtypeStruct((b, m), jnp.float32),
            jax.ShapeDtypeStruct((b, 1), jnp.int32),
            jax.ShapeDtypeStruct((1, m), jnp.int32),
        ],
        scratch_shapes=[pltpu.VMEM((1, m), jnp.int32)],
    )(topk_arr, aff, sam_masks)

    return iou, best2d[:, 0], most2d[0, 0]


# speedup vs baseline: 17.8135x; 1.0675x over previous
import functools

import jax
import jax.numpy as jnp
from jax.experimental import pallas as pl
from jax.experimental.pallas import tpu as pltpu

_TEMPERATURE = 100.0


def _block_kernel(topk_ref, aff_ref, sam_ref, iou_ref, best_ref, most_ref,
                  counts_ref):
    x = aff_ref[...]
    rows, n = x.shape
    m = sam_ref.shape[0]

    logits = x / _TEMPERATURE
    mx = jnp.max(logits, axis=-1, keepdims=True)
    e = jnp.exp(logits - mx)
    denom = jnp.sum(e, axis=-1, keepdims=True)
    sm = e / denom

    keys = jax.lax.bitcast_convert_type(sm, jnp.int32)

    def _row_count(mask):
        v = mask.astype(jnp.int16)
        w = v.shape[1]
        while w > 128:
            w //= 2
            v = v[:, :w] + v[:, w:]
        return jnp.sum(v.astype(jnp.int32), axis=-1, keepdims=True)

    hi = (keys >> 16).astype(jnp.int16)
    k16 = (topk_ref[0] + 1).astype(jnp.int16)

    def body_hi(i, cur):
        cand = cur + (jnp.int32(1) << (jnp.int32(13) - i)).astype(jnp.int16)
        cnt = _row_count(hi >= cand).astype(jnp.int16)
        return jnp.where(cnt >= k16, cand, cur)

    hi_t = jax.lax.fori_loop(0, 14, body_hi,
                             jnp.zeros((rows, 1), jnp.int16))

    cgt = _row_count(hi >= hi_t + jnp.int16(1))
    rk16 = k16 - cgt.astype(jnp.int16)
    tie = hi == hi_t

    lo = ((keys & jnp.int32(0xFFFF)) - jnp.int32(32768)).astype(jnp.int16)

    def body_lo(i, cur):
        cand = (cur.astype(jnp.int32)
                + (jnp.int32(1) << (jnp.int32(15) - i))).astype(jnp.int16)
        cnt = _row_count(tie & (lo >= cand)).astype(jnp.int16)
        return jnp.where(cnt >= rk16, cand, cur)

    lo_t = jax.lax.fori_loop(0, 16, body_lo,
                             jnp.full((rows, 1), jnp.int16(-32768)))

    tkey = ((hi_t.astype(jnp.int32) << 16)
            | ((lo_t.astype(jnp.int32) + 32768) & 0xFFFF))
    thresh = jax.lax.bitcast_convert_type(tkey, jnp.float32)

    sam = sam_ref[...]
    masked = jnp.where(sm > thresh, sm, 0.0)
    inter = jax.lax.dot_general(
        masked, sam, (((1,), (1,)), ((), ())),
        preferred_element_type=jnp.float32)
    sel_sum = jnp.sum(masked, axis=-1, keepdims=True)
    sam_sum = jnp.sum(sam, axis=-1)
    union = sel_sum + sam_sum[None, :] - inter
    iou = inter / (union + 1e-8)
    iou_ref[...] = iou

    iou_mx = jnp.max(iou, axis=-1, keepdims=True)
    lane = jax.lax.broadcasted_iota(jnp.int32, (rows, m), 1)
    best = jnp.min(jnp.where(iou == iou_mx, lane, m), axis=-1)
    best_ref[...] = best[:, None]

    @pl.when(pl.program_id(0) == 0)
    def _init():
        counts_ref[...] = jnp.zeros_like(counts_ref)

    onehot = (best[:, None] == lane).astype(jnp.int32)
    counts_ref[...] += jnp.sum(onehot, axis=0, keepdims=True)

    @pl.when(pl.program_id(0) == pl.num_programs(0) - 1)
    def _finalize():
        c = counts_ref[...]
        c_mx = jnp.max(c, axis=-1, keepdims=True)
        c_lane = jax.lax.broadcasted_iota(jnp.int32, (1, m), 1)
        win = jnp.min(jnp.where(c == c_mx, c_lane, m), axis=-1, keepdims=True)
        most_ref[...] = jnp.broadcast_to(win, (1, m))


def kernel(aff, sam_masks, topk):
    b, n = aff.shape
    m = sam_masks.shape[0]
    block_rows = 256 if b % 256 == 0 else b
    grid = b // block_rows
    topk_arr = jnp.asarray(topk, jnp.int32).reshape(1)

    iou, best2d, most2d = pl.pallas_call(
        _block_kernel,
        grid=(grid,),
        in_specs=[
            pl.BlockSpec(memory_space=pltpu.SMEM),
            pl.BlockSpec((block_rows, n), lambda i: (i, 0)),
            pl.BlockSpec((m, n), lambda i: (0, 0)),
        ],
        out_specs=[
            pl.BlockSpec((block_rows, m), lambda i: (i, 0)),
            pl.BlockSpec((block_rows, 1), lambda i: (i, 0)),
            pl.BlockSpec((1, m), lambda i: (0, 0)),
        ],
        out_shape=[
            jax.ShapeDtypeStruct((b, m), jnp.float32),
            jax.ShapeDtypeStruct((b, 1), jnp.int32),
            jax.ShapeDtypeStruct((1, m), jnp.int32),
        ],
        scratch_shapes=[pltpu.VMEM((1, m), jnp.int32)],
    )(topk_arr, aff, sam_masks)

    return iou, best2d[:, 0], most2d[0, 0]


# speedup vs baseline: 26.5612x; 1.4911x over previous
import functools

import jax
import jax.numpy as jnp
from jax.experimental import pallas as pl
from jax.experimental.pallas import tpu as pltpu

_TEMPERATURE = 100.0


def _block_kernel(topk_ref, aff_ref, sam_ref, iou_ref, best_ref, most_ref,
                  counts_ref):
    x = aff_ref[...]
    rows, n = x.shape
    m = sam_ref.shape[0]

    logits = x / _TEMPERATURE
    mx = jnp.max(logits, axis=-1, keepdims=True)
    e = jnp.exp(logits - mx)
    denom = jnp.sum(e, axis=-1, keepdims=True)
    sm = e / denom

    keys = jax.lax.bitcast_convert_type(sm, jnp.int32)

    def _row_count(mask):
        v = mask.astype(jnp.int16)
        w = v.shape[1]
        while w > 128:
            w //= 2
            v = v[:, :w] + v[:, w:]
        return jnp.sum(v.astype(jnp.int32), axis=-1, keepdims=True)

    hi = (keys >> 16).astype(jnp.int16)
    k16 = (topk_ref[0] + 1).astype(jnp.int16)

    def body_hi(i, cur):
        cand = cur + (jnp.int32(1) << (jnp.int32(13) - i)).astype(jnp.int16)
        cnt = _row_count(hi >= cand).astype(jnp.int16)
        return jnp.where(cnt >= k16, cand, cur)

    hi_t = jax.lax.fori_loop(0, 14, body_hi,
                             jnp.zeros((rows, 1), jnp.int16),
                             unroll=7)

    cgt = _row_count(hi >= hi_t + jnp.int16(1))
    rk16 = k16 - cgt.astype(jnp.int16)
    tie = hi == hi_t

    lo = jnp.where(tie,
                   ((keys & jnp.int32(0xFFFF))
                    - jnp.int32(32768)).astype(jnp.int16),
                   jnp.int16(-32768))

    def body_lo(i, cur):
        cand = (cur.astype(jnp.int32)
                + (jnp.int32(1) << (jnp.int32(15) - i))).astype(jnp.int16)
        cnt = _row_count(lo >= cand).astype(jnp.int16)
        return jnp.where(cnt >= rk16, cand, cur)

    lo_t = jax.lax.fori_loop(0, 16, body_lo,
                             jnp.full((rows, 1), jnp.int16(-32768)),
                             unroll=4)

    tkey = ((hi_t.astype(jnp.int32) << 16)
            | ((lo_t.astype(jnp.int32) + 32768) & 0xFFFF))
    thresh = jax.lax.bitcast_convert_type(tkey, jnp.float32)

    sam = sam_ref[...]
    masked = jnp.where(sm > thresh, sm, 0.0)
    inter = jax.lax.dot_general(
        masked, sam, (((1,), (1,)), ((), ())),
        preferred_element_type=jnp.float32)
    sel_sum = jnp.sum(masked, axis=-1, keepdims=True)
    sam_sum = jnp.sum(sam, axis=-1)
    union = sel_sum + sam_sum[None, :] - inter
    iou = inter / (union + 1e-8)
    iou_ref[...] = iou

    iou_mx = jnp.max(iou, axis=-1, keepdims=True)
    lane = jax.lax.broadcasted_iota(jnp.int32, (rows, m), 1)
    best = jnp.min(jnp.where(iou == iou_mx, lane, m), axis=-1)
    best_ref[...] = best[:, None]

    @pl.when(pl.program_id(0) == 0)
    def _init():
        counts_ref[...] = jnp.zeros_like(counts_ref)

    onehot = (best[:, None] == lane).astype(jnp.int32)
    counts_ref[...] += jnp.sum(onehot, axis=0, keepdims=True)

    @pl.when(pl.program_id(0) == pl.num_programs(0) - 1)
    def _finalize():
        c = counts_ref[...]
        c_mx = jnp.max(c, axis=-1, keepdims=True)
        c_lane = jax.lax.broadcasted_iota(jnp.int32, (1, m), 1)
        win = jnp.min(jnp.where(c == c_mx, c_lane, m), axis=-1, keepdims=True)
        most_ref[...] = jnp.broadcast_to(win, (1, m))


def kernel(aff, sam_masks, topk):
    b, n = aff.shape
    m = sam_masks.shape[0]
    block_rows = 256 if b % 256 == 0 else b
    grid = b // block_rows
    topk_arr = jnp.asarray(topk, jnp.int32).reshape(1)

    iou, best2d, most2d = pl.pallas_call(
        _block_kernel,
        grid=(grid,),
        in_specs=[
            pl.BlockSpec(memory_space=pltpu.SMEM),
            pl.BlockSpec((block_rows, n), lambda i: (i, 0)),
            pl.BlockSpec((m, n), lambda i: (0, 0)),
        ],
        out_specs=[
            pl.BlockSpec((block_rows, m), lambda i: (i, 0)),
            pl.BlockSpec((block_rows, 1), lambda i: (i, 0)),
            pl.BlockSpec((1, m), lambda i: (0, 0)),
        ],
        out_shape=[
            jax.ShapeDtypeStruct((b, m), jnp.float32),
            jax.ShapeDtypeStruct((b, 1), jnp.int32),
            jax.ShapeDtypeStruct((1, m), jnp.int32),
        ],
        scratch_shapes=[pltpu.VMEM((1, m), jnp.int32)],
    )(topk_arr, aff, sam_masks)

    return iou, best2d[:, 0], most2d[0, 0]


# speedup vs baseline: 28.7221x; 1.0814x over previous
import functools

import jax
import jax.numpy as jnp
from jax.experimental import pallas as pl
from jax.experimental.pallas import tpu as pltpu

_TEMPERATURE = 100.0


def _block_kernel(topk_ref, aff_ref, sam_ref, iou_ref, best_ref, most_ref,
                  counts_ref):
    x = aff_ref[...]
    rows, n = x.shape
    m = sam_ref.shape[0]

    logits = x / _TEMPERATURE
    mx = jnp.max(logits, axis=-1, keepdims=True)
    e = jnp.exp(logits - mx)
    denom = jnp.sum(e, axis=-1, keepdims=True)
    sm = e / denom

    keys = jax.lax.bitcast_convert_type(sm, jnp.int32)

    def _row_count(mask):
        v = mask.astype(jnp.int16)
        w = v.shape[1]
        while w > 128:
            w //= 2
            v = v[:, :w] + v[:, w:]
        return jnp.sum(v.astype(jnp.int32), axis=-1, keepdims=True)

    hi = (keys >> 16).astype(jnp.int16)
    k16 = (topk_ref[0] + 1).astype(jnp.int16)

    def body_hi(i, cur):
        cand = cur + (jnp.int32(1) << (jnp.int32(13) - i)).astype(jnp.int16)
        cnt = _row_count(hi >= cand).astype(jnp.int16)
        return jnp.where(cnt >= k16, cand, cur)

    hi_t = jax.lax.fori_loop(0, 14, body_hi,
                             jnp.zeros((rows, 1), jnp.int16),
                             unroll=14)

    cgt = _row_count(hi >= hi_t + jnp.int16(1))
    rk16 = k16 - cgt.astype(jnp.int16)
    tie = hi == hi_t

    lo = jnp.where(tie,
                   ((keys & jnp.int32(0xFFFF))
                    - jnp.int32(32768)).astype(jnp.int16),
                   jnp.int16(-32768))

    def body_lo(i, cur):
        cand = (cur.astype(jnp.int32)
                + (jnp.int32(1) << (jnp.int32(15) - i))).astype(jnp.int16)
        cnt = _row_count(lo >= cand).astype(jnp.int16)
        return jnp.where(cnt >= rk16, cand, cur)

    lo_t = jax.lax.fori_loop(0, 16, body_lo,
                             jnp.full((rows, 1), jnp.int16(-32768)),
                             unroll=16)

    tkey = ((hi_t.astype(jnp.int32) << 16)
            | ((lo_t.astype(jnp.int32) + 32768) & 0xFFFF))
    thresh = jax.lax.bitcast_convert_type(tkey, jnp.float32)

    sam = sam_ref[...]
    masked = jnp.where(sm > thresh, sm, 0.0)
    inter = jax.lax.dot_general(
        masked, sam, (((1,), (1,)), ((), ())),
        preferred_element_type=jnp.float32)
    sel_sum = jnp.sum(masked, axis=-1, keepdims=True)
    sam_sum = jnp.sum(sam, axis=-1)
    union = sel_sum + sam_sum[None, :] - inter
    iou = inter / (union + 1e-8)
    iou_ref[...] = iou

    iou_mx = jnp.max(iou, axis=-1, keepdims=True)
    lane = jax.lax.broadcasted_iota(jnp.int32, (rows, m), 1)
    best = jnp.min(jnp.where(iou == iou_mx, lane, m), axis=-1)
    best_ref[...] = best[:, None]

    @pl.when(pl.program_id(0) == 0)
    def _init():
        counts_ref[...] = jnp.zeros_like(counts_ref)

    onehot = (best[:, None] == lane).astype(jnp.int32)
    counts_ref[...] += jnp.sum(onehot, axis=0, keepdims=True)

    @pl.when(pl.program_id(0) == pl.num_programs(0) - 1)
    def _finalize():
        c = counts_ref[...]
        c_mx = jnp.max(c, axis=-1, keepdims=True)
        c_lane = jax.lax.broadcasted_iota(jnp.int32, (1, m), 1)
        win = jnp.min(jnp.where(c == c_mx, c_lane, m), axis=-1, keepdims=True)
        most_ref[...] = jnp.broadcast_to(win, (1, m))


def kernel(aff, sam_masks, topk):
    b, n = aff.shape
    m = sam_masks.shape[0]
    block_rows = 256 if b % 256 == 0 else b
    grid = b // block_rows
    topk_arr = jnp.asarray(topk, jnp.int32).reshape(1)

    iou, best2d, most2d = pl.pallas_call(
        _block_kernel,
        grid=(grid,),
        in_specs=[
            pl.BlockSpec(memory_space=pltpu.SMEM),
            pl.BlockSpec((block_rows, n), lambda i: (i, 0)),
            pl.BlockSpec((m, n), lambda i: (0, 0)),
        ],
        out_specs=[
            pl.BlockSpec((block_rows, m), lambda i: (i, 0)),
            pl.BlockSpec((block_rows, 1), lambda i: (i, 0)),
            pl.BlockSpec((1, m), lambda i: (0, 0)),
        ],
        out_shape=[
            jax.ShapeDtypeStruct((b, m), jnp.float32),
            jax.ShapeDtypeStruct((b, 1), jnp.int32),
            jax.ShapeDtypeStruct((1, m), jnp.int32),
        ],
        scratch_shapes=[pltpu.VMEM((1, m), jnp.int32)],
    )(topk_arr, aff, sam_masks)

    return iou, best2d[:, 0], most2d[0, 0]


# speedup vs baseline: 28.9904x; 1.0093x over previous
import functools

import jax
import jax.numpy as jnp
from jax.experimental import pallas as pl
from jax.experimental.pallas import tpu as pltpu

_TEMPERATURE = 100.0


def _block_kernel(topk_ref, aff_ref, sam_ref, iou_ref, best_ref, most_ref,
                  counts_ref):
    x = aff_ref[...]
    rows, n = x.shape
    m = sam_ref.shape[0]

    logits = x / _TEMPERATURE
    mx = jnp.max(logits, axis=-1, keepdims=True)
    e = jnp.exp(logits - mx)
    denom = jnp.sum(e, axis=-1, keepdims=True)
    sm = e / denom

    keys = jax.lax.bitcast_convert_type(sm, jnp.int32)

    def _row_count(mask):
        v = mask.astype(jnp.int16)
        w = v.shape[1]
        while w > 128:
            w //= 2
            v = v[:, :w] + v[:, w:]
        return jnp.sum(v.astype(jnp.int32), axis=-1, keepdims=True)

    hi = (keys >> 16).astype(jnp.int16)
    k16 = (topk_ref[0] + 1).astype(jnp.int16)

    def body_hi(i, cur):
        cand = cur + (jnp.int32(1) << (jnp.int32(13) - i)).astype(jnp.int16)
        cnt = _row_count(hi >= cand).astype(jnp.int16)
        return jnp.where(cnt >= k16, cand, cur)

    hi_t = jax.lax.fori_loop(0, 14, body_hi,
                             jnp.zeros((rows, 1), jnp.int16),
                             unroll=14)

    cgt = _row_count(hi >= hi_t + jnp.int16(1))
    rk16 = k16 - cgt.astype(jnp.int16)
    tie = hi == hi_t

    lo = jnp.where(tie,
                   ((keys & jnp.int32(0xFFFF))
                    - jnp.int32(32768)).astype(jnp.int16),
                   jnp.int16(-32768))

    def body_lo(i, cur):
        cand = (cur.astype(jnp.int32)
                + (jnp.int32(1) << (jnp.int32(15) - i))).astype(jnp.int16)
        cnt = _row_count(lo >= cand).astype(jnp.int16)
        return jnp.where(cnt >= rk16, cand, cur)

    lo_t = jax.lax.fori_loop(0, 16, body_lo,
                             jnp.full((rows, 1), jnp.int16(-32768)),
                             unroll=16)

    tkey = ((hi_t.astype(jnp.int32) << 16)
            | ((lo_t.astype(jnp.int32) + 32768) & 0xFFFF))
    thresh = jax.lax.bitcast_convert_type(tkey, jnp.float32)

    sam = sam_ref[...]
    masked = jnp.where(sm > thresh, sm, 0.0)
    inter = jax.lax.dot_general(
        masked, sam, (((1,), (1,)), ((), ())),
        preferred_element_type=jnp.float32)
    sel_sum = jnp.sum(masked, axis=-1, keepdims=True)
    sam_sum = jnp.sum(sam, axis=-1)
    union = sel_sum + sam_sum[None, :] - inter
    iou = inter / (union + 1e-8)
    iou_ref[...] = iou

    iou_mx = jnp.max(iou, axis=-1, keepdims=True)
    lane = jax.lax.broadcasted_iota(jnp.int32, (rows, m), 1)
    best = jnp.min(jnp.where(iou == iou_mx, lane, m), axis=-1)
    best_ref[...] = best[:, None]

    @pl.when(pl.program_id(0) == 0)
    def _init():
        counts_ref[...] = jnp.zeros_like(counts_ref)

    onehot = (best[:, None] == lane).astype(jnp.int32)
    counts_ref[...] += jnp.sum(onehot, axis=0, keepdims=True)

    @pl.when(pl.program_id(0) == pl.num_programs(0) - 1)
    def _finalize():
        c = counts_ref[...]
        c_mx = jnp.max(c, axis=-1, keepdims=True)
        c_lane = jax.lax.broadcasted_iota(jnp.int32, (1, m), 1)
        win = jnp.min(jnp.where(c == c_mx, c_lane, m), axis=-1, keepdims=True)
        most_ref[...] = jnp.broadcast_to(win, (1, m))


def kernel(aff, sam_masks, topk):
    b, n = aff.shape
    m = sam_masks.shape[0]
    block_rows = 512 if b % 512 == 0 else b
    grid = b // block_rows
    topk_arr = jnp.asarray(topk, jnp.int32).reshape(1)

    iou, best2d, most2d = pl.pallas_call(
        _block_kernel,
        grid=(grid,),
        in_specs=[
            pl.BlockSpec(memory_space=pltpu.SMEM),
            pl.BlockSpec((block_rows, n), lambda i: (i, 0)),
            pl.BlockSpec((m, n), lambda i: (0, 0)),
        ],
        out_specs=[
            pl.BlockSpec((block_rows, m), lambda i: (i, 0)),
            pl.BlockSpec((block_rows, 1), lambda i: (i, 0)),
            pl.BlockSpec((1, m), lambda i: (0, 0)),
        ],
        out_shape=[
            jax.ShapeDtypeStruct((b, m), jnp.float32),
            jax.ShapeDtypeStruct((b, 1), jnp.int32),
            jax.ShapeDtypeStruct((1, m), jnp.int32),
        ],
        scratch_shapes=[pltpu.VMEM((1, m), jnp.int32)],
    )(topk_arr, aff, sam_masks)

    return iou, best2d[:, 0], most2d[0, 0]
